# baseline (speedup 1.0000x reference)
import functools

import jax
import jax.numpy as jnp
from jax import lax
from jax.experimental import pallas as pl
from jax.experimental.pallas import tpu as pltpu
from jax.experimental.pallas import tpu_sc as plsc

N = 10000
DH = 256
W = 144
NC, NS, L = 2, 16, 16
C = 80
E = 320000
NCHUNK = E // (NS * C)
EPW = E // (NC * NS)
ACC_ROWS = NS * 640
RPW = 632
OUT_ROWS = NS * RPW

_SC_PARAMS = pltpu.CompilerParams(
    needs_layout_passes=False, use_tc_tiling_on_sc=False
)
_MESH = plsc.VectorSubcoreMesh(
    core_axis_name="c", subcore_axis_name="s", num_cores=NC, num_subcores=NS
)


def _mm_kernel(x_ref, w_ref, b_ref, o_ref, *, act):
    y = jnp.dot(x_ref[...], w_ref[...], preferred_element_type=jnp.float32)
    y = y + b_ref[...]
    if act == "relu":
        y = jnp.maximum(y, 0.0)
    o_ref[...] = y


def _mm(x, w, b, act=None, bm=2000):
    m, k = x.shape
    n = w.shape[1]
    return pl.pallas_call(
        functools.partial(_mm_kernel, act=act),
        grid=(m // bm,),
        in_specs=[
            pl.BlockSpec((bm, k), lambda i: (i, 0)),
            pl.BlockSpec((k, n), lambda i: (0, 0)),
            pl.BlockSpec((1, n), lambda i: (0, 0)),
        ],
        out_specs=pl.BlockSpec((bm, n), lambda i: (i, 0)),
        out_shape=jax.ShapeDtypeStruct((m, n), jnp.float32),
    )(x, w, b.reshape(1, n))


def _comb_kernel(a0_ref, a1_ref, xa1_ref, xa2_ref, x_ref, o_ref):
    v = xa1_ref[...] + xa2_ref[...]
    w2 = jnp.exp(jnp.where(v >= 0, v, 0.2 * v))
    agg = jnp.concatenate([a0_ref[...], a1_ref[...][:, : DH - W]], axis=1)
    div = a1_ref[...][:, DH - W : DH - W + 1] + w2
    y = (agg + w2 * x_ref[...]) / div
    o_ref[...] = jnp.where(y >= 0, y, jnp.exp(y) - 1.0)


def _combine(a0, a1, xa1, xa2, x, bm=2000):
    return pl.pallas_call(
        _comb_kernel,
        grid=(N // bm,),
        in_specs=[
            pl.BlockSpec((bm, W), lambda i: (i, 0)),
            pl.BlockSpec((bm, W), lambda i: (i, 0)),
            pl.BlockSpec((bm, 1), lambda i: (i, 0)),
            pl.BlockSpec((bm, 1), lambda i: (i, 0)),
            pl.BlockSpec((bm, DH), lambda i: (i, 0)),
        ],
        out_specs=pl.BlockSpec((bm, DH), lambda i: (i, 0)),
        out_shape=jax.ShapeDtypeStruct((N, DH), jnp.float32),
    )(a0, a1, xa1, xa2, x)


def _w_pass(su, tu, si, ti, x1u, h1i, x1i, h1u):
    @functools.partial(
        pl.kernel,
        out_type=(
            jax.ShapeDtypeStruct((E,), jnp.float32),
            jax.ShapeDtypeStruct((E,), jnp.float32),
        ),
        mesh=_MESH,
        compiler_params=_SC_PARAMS,
        scratch_types=[
            pltpu.VMEM((N,), jnp.float32),
            pltpu.VMEM((N,), jnp.float32),
            pltpu.VMEM((N,), jnp.float32),
            pltpu.VMEM((N,), jnp.float32),
            pltpu.VMEM((EPW,), jnp.int32),
            pltpu.VMEM((EPW,), jnp.int32),
            pltpu.VMEM((EPW,), jnp.float32),
        ],
    )
    def k(su_h, tu_h, si_h, ti_h, x1u_h, h1i_h, x1i_h, h1u_h, wu_h, wi_h,
          x1u_v, h1i_v, x1i_v, h1u_v, sbuf, tbuf, wbuf):
        c = lax.axis_index("c")
        sid = lax.axis_index("s")
        base = (sid * NC + c) * EPW
        pltpu.sync_copy(x1u_h, x1u_v)
        pltpu.sync_copy(h1i_h, h1i_v)
        pltpu.sync_copy(x1i_h, x1i_v)
        pltpu.sync_copy(h1u_h, h1u_v)
        for (s_h, t_h, x1_v, h1_v, w_h) in (
            (su_h, tu_h, x1u_v, h1i_v, wu_h),
            (si_h, ti_h, x1i_v, h1u_v, wi_h),
        ):
            pltpu.sync_copy(s_h.at[pl.ds(base, EPW)], sbuf)
            pltpu.sync_copy(t_h.at[pl.ds(base, EPW)], tbuf)

            def grp(g, carry, x1_v=x1_v, h1_v=h1_v):
                sl = pl.ds(g * L, L)
                v = plsc.load_gather(x1_v, [sbuf[sl]]) + plsc.load_gather(
                    h1_v, [tbuf[sl]]
                )
                wbuf[sl] = jnp.exp(jnp.where(v >= 0, v, 0.2 * v))
                return carry

            lax.fori_loop(0, EPW // L, grp, 0)
            pltpu.sync_copy(wbuf, w_h.at[pl.ds(base, EPW)])

    return k(su, tu, si, ti, x1u, h1i, x1i, h1u)


def _agg_pass(s, tadj, w, table):
    @functools.partial(
        pl.kernel,
        out_type=jax.ShapeDtypeStruct((NC, OUT_ROWS, W), jnp.float32),
        mesh=_MESH,
        compiler_params=_SC_PARAMS,
        scratch_types=[
            pltpu.VMEM((3, C), jnp.int32),
            pltpu.VMEM((3, C), jnp.int32),
            pltpu.VMEM((3, C), jnp.float32),
            pltpu.VMEM((3, C, W), jnp.float32),
            pltpu.VMEM_SHARED((ACC_ROWS, W), jnp.float32),
            pltpu.SemaphoreType.DMA((3,)),
            pltpu.SemaphoreType.DMA((3,)),
            pltpu.SemaphoreType.DMA((3,)),
        ],
    )
    def k(s_hbm, tadj_hbm, w_hbm, tbl_hbm, out_hbm,
          sidx3, gidx3, w3, rows3, acc, msem, gsem, ssem):
        c = lax.axis_index("c")
        sid = lax.axis_index("s")
        base = sid * (NCHUNK * C)

        def zr(r, carry):
            for cg in range(W // L):
                rows3[0, r, pl.ds(cg * L, L)] = jnp.zeros((L,), jnp.float32)
            return carry

        lax.fori_loop(0, C, zr, 0)

        def za(kk, carry):
            pltpu.sync_copy(rows3.at[0], acc.at[pl.ds(sid * 640 + kk * C, C)])
            return carry

        lax.fori_loop(0, 640 // C, za, 0)
        plsc.subcore_barrier()

        def meta_copies(m, bi):
            st = base + m * C
            return (
                (s_hbm.at[pl.ds(st, C)], sidx3.at[bi]),
                (tadj_hbm.at[c, pl.ds(st, C)], gidx3.at[bi]),
                (w_hbm.at[pl.ds(st, C)], w3.at[bi]),
            )

        def issue_meta(m, bi):
            for src, dst in meta_copies(m, bi):
                pltpu.async_copy(src, dst, msem.at[bi])

        def wait_meta(m, bi):
            for src, dst in meta_copies(m, bi):
                pltpu.make_async_copy(src, dst, msem.at[bi]).wait()

        def issue_gather(bi):
            pltpu.async_copy(tbl_hbm.at[gidx3.at[bi]], rows3.at[bi], gsem.at[bi])

        def wait_gather(bi):
            pltpu.make_async_copy(
                tbl_hbm.at[gidx3.at[bi]], rows3.at[bi], gsem.at[bi]
            ).wait()

        def issue_scatter(bi):
            pltpu.async_copy(
                rows3.at[bi], acc.at[sidx3.at[bi]], ssem.at[bi], add=True
            )

        def wait_scatter(bi):
            pltpu.make_async_copy(
                rows3.at[bi], acc.at[sidx3.at[bi]], ssem.at[bi]
            ).wait()

        issue_meta(0, 0)
        issue_meta(1, 1)
        wait_meta(0, 0)
        issue_gather(0)

        def body(m, carry):
            bi = m % 3
            gb = (m + 1) % 3
            nb = (m + 2) % 3

            @pl.when(m + 2 < NCHUNK)
            def _():
                @pl.when(m >= 1)
                def _():
                    wait_scatter(nb)

                issue_meta(m + 2, nb)

            @pl.when(m + 1 < NCHUNK)
            def _():
                wait_meta(m + 1, gb)
                issue_gather(gb)

            wait_gather(bi)

            def scale(g, carry2):
                wv = w3[bi, pl.ds(g * L, L)]
                for j in range(L):
                    e = g * L + j
                    we = wv[j]
                    for cg in range(W // L):
                        slc = pl.ds(cg * L, L)
                        rows3[bi, e, slc] = rows3[bi, e, slc] * we
                return carry2

            lax.fori_loop(0, C // L, scale, 0)
            issue_scatter(bi)
            return carry

        lax.fori_loop(0, NCHUNK, body, 0)
        wait_scatter(0)
        wait_scatter(1)
        wait_scatter(2)
        plsc.subcore_barrier()
        pltpu.sync_copy(
            acc.at[pl.ds(sid * RPW, RPW)], out_hbm.at[c, pl.ds(sid * RPW, RPW)]
        )

    return k(s, tadj, w, table)


def _mk_table(x):
    ones = jnp.ones((N, 1), jnp.float32)
    zpad = jnp.zeros((N, W - (DH - W) - 1), jnp.float32)
    hi = jnp.concatenate([x[:, W:DH], ones, zpad], axis=1)
    return jnp.concatenate([x[:, :W], hi], axis=0)


def kernel(x_user, x_item, params, edge_ui, edge_iu):
    p = params
    f32 = jnp.float32
    xu = _mm(x_user, p["W1_user"], p["b1_user"], act="relu")
    xi = _mm(x_item, p["W1_item"], p["b1_item"], act="relu")

    def prep(ei):
        s = ei[0].astype(jnp.int32)
        t = ei[1].astype(jnp.int32)
        tadj = jnp.stack([t, t + N])
        return s, t, tadj

    s_ui, t_ui, tadj_ui = prep(edge_ui)
    s_iu, t_iu, tadj_iu = prep(edge_iu)
    z3 = jnp.zeros((3,), f32)

    for h in range(2):
        xu = _mm(xu, p["Wfc%d" % h], p["bfc%d" % h])
        xi = _mm(xi, p["Wfc%d" % h], p["bfc%d" % h])
        au = jnp.concatenate(
            [p["a1_%d_user_item" % h], p["a2_%d_user_item" % h],
             p["a2_%d_item_user" % h]], axis=1)
        ai = jnp.concatenate(
            [p["a2_%d_user_item" % h], p["a1_%d_item_user" % h],
             p["a2_%d_item_user" % h]], axis=1)
        pu = _mm(xu, au, z3)
        pi = _mm(xi, ai, z3)

        w_ui, w_iu = _w_pass(
            s_ui, t_ui, s_iu, t_iu, pu[:, 0], pi[:, 0], pi[:, 1], pu[:, 2]
        )
        out_ui = _agg_pass(s_ui, tadj_ui, w_ui, _mk_table(xi))
        out_iu = _agg_pass(s_iu, tadj_iu, w_iu, _mk_table(xu))

        xu = _combine(out_ui[0, :N], out_ui[1, :N], pu[:, 0:1], pu[:, 1:2], xu)
        xi = _combine(out_iu[0, :N], out_iu[1, :N], pi[:, 1:2], pi[:, 2:3], xi)

    return _mm(xu, p["Wout"], p["bout"])


# speedup vs baseline: 1.7676x; 1.7676x over previous
import functools

import jax
import jax.numpy as jnp
from jax import lax
from jax.experimental import pallas as pl
from jax.experimental.pallas import tpu as pltpu
from jax.experimental.pallas import tpu_sc as plsc

N = 10000
DH = 256
W = 144
NC, NS, L = 2, 16, 16
C = 80
E = 320000
NCHUNK = 252
E_PAD = NS * C * NCHUNK
EPW = E // (NC * NS)
ACC_ROWS = NS * 640
RPW = 632
OUT_ROWS = NS * RPW

_SC_PARAMS = pltpu.CompilerParams(
    needs_layout_passes=False, use_tc_tiling_on_sc=False
)
_MESH = plsc.VectorSubcoreMesh(
    core_axis_name="c", subcore_axis_name="s", num_cores=NC, num_subcores=NS
)


def _mm_kernel(x_ref, w_ref, b_ref, o_ref, *, act):
    y = jnp.dot(x_ref[...], w_ref[...], preferred_element_type=jnp.float32)
    y = y + b_ref[...]
    if act == "relu":
        y = jnp.maximum(y, 0.0)
    o_ref[...] = y


def _mm(x, w, b, act=None, bm=2000):
    m, k = x.shape
    n = w.shape[1]
    return pl.pallas_call(
        functools.partial(_mm_kernel, act=act),
        grid=(m // bm,),
        in_specs=[
            pl.BlockSpec((bm, k), lambda i: (i, 0)),
            pl.BlockSpec((k, n), lambda i: (0, 0)),
            pl.BlockSpec((1, n), lambda i: (0, 0)),
        ],
        out_specs=pl.BlockSpec((bm, n), lambda i: (i, 0)),
        out_shape=jax.ShapeDtypeStruct((m, n), jnp.float32),
    )(x, w, b.reshape(1, n))


def _comb_kernel(a0_ref, a1_ref, xa1_ref, xa2_ref, x_ref, o_ref):
    v = xa1_ref[...] + xa2_ref[...]
    w2 = jnp.exp(jnp.where(v >= 0, v, 0.2 * v))
    agg = jnp.concatenate([a0_ref[...], a1_ref[...][:, : DH - W]], axis=1)
    div = a1_ref[...][:, DH - W : DH - W + 1] + w2
    y = (agg + w2 * x_ref[...]) / div
    o_ref[...] = jnp.where(y >= 0, y, jnp.exp(y) - 1.0)


def _combine(a0, a1, xa1, xa2, x, bm=2000):
    return pl.pallas_call(
        _comb_kernel,
        grid=(N // bm,),
        in_specs=[
            pl.BlockSpec((bm, W), lambda i: (i, 0)),
            pl.BlockSpec((bm, W), lambda i: (i, 0)),
            pl.BlockSpec((bm, 1), lambda i: (i, 0)),
            pl.BlockSpec((bm, 1), lambda i: (i, 0)),
            pl.BlockSpec((bm, DH), lambda i: (i, 0)),
        ],
        out_specs=pl.BlockSpec((bm, DH), lambda i: (i, 0)),
        out_shape=jax.ShapeDtypeStruct((N, DH), jnp.float32),
    )(a0, a1, xa1, xa2, x)


def _w_pass(su, tu, si, ti, x1u, h1i, x1i, h1u):
    @functools.partial(
        pl.kernel,
        out_type=(
            jax.ShapeDtypeStruct((E,), jnp.float32),
            jax.ShapeDtypeStruct((E,), jnp.float32),
        ),
        mesh=_MESH,
        compiler_params=_SC_PARAMS,
        scratch_types=[
            pltpu.VMEM((N,), jnp.float32),
            pltpu.VMEM((N,), jnp.float32),
            pltpu.VMEM((N,), jnp.float32),
            pltpu.VMEM((N,), jnp.float32),
            pltpu.VMEM((EPW,), jnp.int32),
            pltpu.VMEM((EPW,), jnp.int32),
            pltpu.VMEM((EPW,), jnp.float32),
        ],
    )
    def k(su_h, tu_h, si_h, ti_h, x1u_h, h1i_h, x1i_h, h1u_h, wu_h, wi_h,
          x1u_v, h1i_v, x1i_v, h1u_v, sbuf, tbuf, wbuf):
        c = lax.axis_index("c")
        sid = lax.axis_index("s")
        base = (sid * NC + c) * EPW
        pltpu.sync_copy(x1u_h, x1u_v)
        pltpu.sync_copy(h1i_h, h1i_v)
        pltpu.sync_copy(x1i_h, x1i_v)
        pltpu.sync_copy(h1u_h, h1u_v)
        for (s_h, t_h, x1_v, h1_v, w_h) in (
            (su_h, tu_h, x1u_v, h1i_v, wu_h),
            (si_h, ti_h, x1i_v, h1u_v, wi_h),
        ):
            pltpu.sync_copy(s_h.at[pl.ds(base, EPW)], sbuf)
            pltpu.sync_copy(t_h.at[pl.ds(base, EPW)], tbuf)

            def grp(g, carry, x1_v=x1_v, h1_v=h1_v):
                sl = pl.ds(g * L, L)
                v = plsc.load_gather(x1_v, [sbuf[sl]]) + plsc.load_gather(
                    h1_v, [tbuf[sl]]
                )
                wbuf[sl] = jnp.exp(jnp.where(v >= 0, v, 0.2 * v))
                return carry

            lax.fori_loop(0, EPW // L, grp, 0)
            pltpu.sync_copy(wbuf, w_h.at[pl.ds(base, EPW)])

    return k(su, tu, si, ti, x1u, h1i, x1i, h1u)


def _agg_pass(meta, w, table):
    @functools.partial(
        pl.kernel,
        out_type=jax.ShapeDtypeStruct((NC, OUT_ROWS, W), jnp.float32),
        mesh=_MESH,
        compiler_params=_SC_PARAMS,
        scratch_types=[
            pltpu.VMEM((3, 3, C), jnp.int32),
            pltpu.VMEM((3, C), jnp.float32),
            pltpu.VMEM((3, C, W), jnp.float32),
            pltpu.VMEM_SHARED((ACC_ROWS, W), jnp.float32),
            pltpu.SemaphoreType.DMA((3,)),
            pltpu.SemaphoreType.DMA((3,)),
            pltpu.SemaphoreType.DMA((3,)),
        ],
    )
    def k(meta_hbm, w_hbm, tbl_hbm, out_hbm,
          meta3, w3, rows3, acc, msem, gsem, ssem):
        c = lax.axis_index("c")
        sid = lax.axis_index("s")
        ebase = sid * (NCHUNK * C)
        mbase = sid * NCHUNK

        def zr(r, carry):
            for cg in range(W // L):
                rows3[0, r, pl.ds(cg * L, L)] = jnp.zeros((L,), jnp.float32)
            return carry

        lax.fori_loop(0, C, zr, 0)

        def za(kk, carry):
            pltpu.sync_copy(rows3.at[0], acc.at[pl.ds(sid * 640 + kk * C, C)])
            return carry

        lax.fori_loop(0, 640 // C, za, 0)
        plsc.subcore_barrier()

        def meta_copies(m, bi):
            return (
                (meta_hbm.at[mbase + m], meta3.at[bi]),
                (w_hbm.at[pl.ds(ebase + m * C, C)], w3.at[bi]),
            )

        def issue_meta(m, bi):
            for src, dst in meta_copies(m, bi):
                pltpu.async_copy(src, dst, msem.at[bi])

        def wait_meta(m, bi):
            for src, dst in meta_copies(m, bi):
                pltpu.make_async_copy(src, dst, msem.at[bi]).wait()

        def issue_gather(bi):
            pltpu.async_copy(
                tbl_hbm.at[meta3.at[bi, 1 + c]], rows3.at[bi], gsem.at[bi]
            )

        def wait_gather(bi):
            pltpu.make_async_copy(
                tbl_hbm.at[meta3.at[bi, 1 + c]], rows3.at[bi], gsem.at[bi]
            ).wait()

        def issue_scatter(bi):
            pltpu.async_copy(
                rows3.at[bi], acc.at[meta3.at[bi, 0]], ssem.at[bi], add=True
            )

        def wait_scatter(bi):
            pltpu.make_async_copy(
                rows3.at[bi], acc.at[meta3.at[bi, 0]], ssem.at[bi]
            ).wait()

        def scale(bi):
            def grp(g, carry2):
                wv = w3[bi, pl.ds(g * L, L)]
                for j in range(L):
                    e = g * L + j
                    we = wv[j]
                    for cg in range(W // L):
                        slc = pl.ds(cg * L, L)
                        rows3[bi, e, slc] = rows3[bi, e, slc] * we
                return carry2

            lax.fori_loop(0, C // L, grp, 0)

        def step(m, ph, do_meta, do_gather, do_scatter_wait):
            gb, nb = (ph + 1) % 3, (ph + 2) % 3
            if do_meta:
                if do_scatter_wait:
                    wait_scatter(nb)
                issue_meta(m + 2, nb)
            if do_gather:
                wait_meta(m + 1, gb)
                issue_gather(gb)
            wait_gather(ph)
            scale(ph)
            issue_scatter(ph)

        issue_meta(0, 0)
        issue_meta(1, 1)
        wait_meta(0, 0)
        issue_gather(0)

        step(0, 0, True, True, False)
        step(1, 1, True, True, True)
        step(2, 2, True, True, True)

        def body(kk, carry):
            m0 = kk * 3
            step(m0, 0, True, True, True)
            step(m0 + 1, 1, True, True, True)
            step(m0 + 2, 2, True, True, True)
            return carry

        lax.fori_loop(1, NCHUNK // 3 - 1, body, 0)

        step(NCHUNK - 3, 0, True, True, True)
        step(NCHUNK - 2, 1, False, True, False)
        step(NCHUNK - 1, 2, False, False, False)

        wait_scatter(0)
        wait_scatter(1)
        wait_scatter(2)
        plsc.subcore_barrier()
        pltpu.sync_copy(
            acc.at[pl.ds(sid * RPW, RPW)], out_hbm.at[c, pl.ds(sid * RPW, RPW)]
        )

    return k(meta, w, table)


def _mk_table(x):
    ones = jnp.ones((N, 1), jnp.float32)
    zpad = jnp.zeros((N, W - (DH - W) - 1), jnp.float32)
    hi = jnp.concatenate([x[:, W:DH], ones, zpad], axis=1)
    return jnp.concatenate([x[:, :W], hi], axis=0)


def kernel(x_user, x_item, params, edge_ui, edge_iu):
    p = params
    f32 = jnp.float32
    xu = _mm(x_user, p["W1_user"], p["b1_user"], act="relu")
    xi = _mm(x_item, p["W1_item"], p["b1_item"], act="relu")

    npad = E_PAD - E

    def prep(ei):
        s = ei[0].astype(jnp.int32)
        t = ei[1].astype(jnp.int32)
        sp = jnp.concatenate([s, jnp.full((npad,), N, jnp.int32)])
        t0 = jnp.concatenate([t, jnp.zeros((npad,), jnp.int32)])
        t1 = jnp.concatenate([t + N, jnp.zeros((npad,), jnp.int32)])
        meta = jnp.stack(
            [x.reshape(NS, NCHUNK, C) for x in (sp, t0, t1)], axis=2
        ).reshape(NS * NCHUNK, 3, C)
        return s, t, meta

    s_ui, t_ui, meta_ui = prep(edge_ui)
    s_iu, t_iu, meta_iu = prep(edge_iu)
    z3 = jnp.zeros((3,), f32)

    for h in range(2):
        xu = _mm(xu, p["Wfc%d" % h], p["bfc%d" % h])
        xi = _mm(xi, p["Wfc%d" % h], p["bfc%d" % h])
        au = jnp.concatenate(
            [p["a1_%d_user_item" % h], p["a2_%d_user_item" % h],
             p["a2_%d_item_user" % h]], axis=1)
        ai = jnp.concatenate(
            [p["a2_%d_user_item" % h], p["a1_%d_item_user" % h],
             p["a2_%d_item_user" % h]], axis=1)
        pu = _mm(xu, au, z3)
        pi = _mm(xi, ai, z3)

        w_ui, w_iu = _w_pass(
            s_ui, t_ui, s_iu, t_iu, pu[:, 0], pi[:, 0], pi[:, 1], pu[:, 2]
        )
        out_ui = _agg_pass(meta_ui, jnp.pad(w_ui, (0, npad)), _mk_table(xi))
        out_iu = _agg_pass(meta_iu, jnp.pad(w_iu, (0, npad)), _mk_table(xu))

        xu = _combine(out_ui[0, :N], out_ui[1, :N], pu[:, 0:1], pu[:, 1:2], xu)
        xi = _combine(out_iu[0, :N], out_iu[1, :N], pi[:, 1:2], pi[:, 2:3], xi)

    return _mm(xu, p["Wout"], p["bout"])
